# baseline (speedup 1.0000x reference)
import functools

import jax
import jax.numpy as jnp
from jax import lax
from jax.experimental import pallas as pl
from jax.experimental.pallas import tpu as pltpu
from jax.experimental.pallas import tpu_sc as plsc

_BATCH = 16384
_EMB = 64
_L = 16
_NC = 2
_NS = 16
_NW = _NC * _NS
_N_PER = _BATCH // _NW


def _sc_body(u_hbm, v_hbm, table_hbm, out_hbm,
             uidx_v, vidx_v, urows_v, vrows_v, out_v, sem_u, sem_v):
    wid = lax.axis_index("s") * _NC + lax.axis_index("c")
    base = wid * _N_PER

    pltpu.sync_copy(u_hbm.at[pl.ds(base, _N_PER)], uidx_v)
    pltpu.sync_copy(v_hbm.at[pl.ds(base, _N_PER)], vidx_v)

    cu = pltpu.async_copy(table_hbm.at[uidx_v], urows_v, sem_u)
    cv = pltpu.async_copy(table_hbm.at[vidx_v], vrows_v, sem_v)
    cu.wait()
    cv.wait()

    lane = lax.iota(jnp.int32, _L)

    def group(g, carry):
        rows = g * _L + lane
        acc = jnp.zeros((_L,), jnp.float32)
        for j in range(_EMB):
            col = jnp.full((_L,), j, jnp.int32)
            uu = plsc.load_gather(urows_v, [rows, col])
            vv = plsc.load_gather(vrows_v, [rows, col])
            acc = acc + uu * vv
        out_v[pl.ds(g * _L, _L)] = 1.0 / (1.0 + jnp.exp(-acc))
        return carry

    lax.fori_loop(0, _N_PER // _L, group, 0)

    pltpu.sync_copy(out_v, out_hbm.at[pl.ds(base, _N_PER)])


@jax.jit
def _dist_mult(u, v, emb_weight):
    mesh = plsc.VectorSubcoreMesh(
        core_axis_name="c", subcore_axis_name="s",
        num_cores=_NC, num_subcores=_NS)
    run = pl.kernel(
        _sc_body,
        out_type=jax.ShapeDtypeStruct((_BATCH,), jnp.float32),
        mesh=mesh,
        scratch_types=[
            pltpu.VMEM((_N_PER,), jnp.int32),
            pltpu.VMEM((_N_PER,), jnp.int32),
            pltpu.VMEM((_N_PER, _EMB), jnp.float32),
            pltpu.VMEM((_N_PER, _EMB), jnp.float32),
            pltpu.VMEM((_N_PER,), jnp.float32),
            pltpu.SemaphoreType.DMA,
            pltpu.SemaphoreType.DMA,
        ],
        compiler_params=pltpu.CompilerParams(
            needs_layout_passes=False, use_tc_tiling_on_sc=False),
    )
    return run(u, v, emb_weight)


def kernel(u, v, emb_weight):
    return _dist_mult(u.astype(jnp.int32), v.astype(jnp.int32), emb_weight)
